# Initial kernel scaffold; baseline (speedup 1.0000x reference)
import jax
import jax.numpy as jnp
from jax.experimental import pallas as pl


def kernel(x, edge_index, edge_attr, W1, b1, W2, b2, Wr1, br1, Wr2, br2):
    raise NotImplementedError("write your pallas kernel here")


# speedup vs baseline: 14.9092x; 14.9092x over previous
import functools

import jax
import jax.numpy as jnp
from jax import lax
from jax.experimental import pallas as pl
from jax.experimental.pallas import tpu as pltpu
from jax.experimental.pallas import tpu_sc as plsc

N = 10000
E = 320000
D = 128
NPAD = 10240
CH = 128
NCHUNKS = E // CH
NC, NS = 2, 16
ROWS_PER_SUB = N // NS
PADROWS_PER_SUB = NPAD // NS


def _zero_vec16(ref, nwords):
    z = jnp.zeros((16,), jnp.float32)

    def body(i, _):
        ref[pl.ds(i * 16, 16)] = z
        return 0

    lax.fori_loop(0, nwords // 16, body, 0)


def _zero_rows(ref, nrows):
    z = jnp.zeros((16,), jnp.float32)

    def body(r, _):
        for dblk in range(8):
            ref[r, pl.ds(dblk * 16, 16)] = z
        return 0

    lax.fori_loop(0, nrows, body, 0)


def _rsqrt16(d):
    m = d
    y = jnp.full((16,), 1.0, jnp.float32)
    for _ in range(10):
        c = m > 4.0
        m = jnp.where(c, m * 0.25, m)
        y = jnp.where(c, y * 0.5, y)
    r = jnp.full((16,), 0.7, jnp.float32)
    for _ in range(5):
        r = r * (1.5 - 0.5 * m * r * r)
    return y * r


def _aggregate_edges(row_hbm, col_hbm, ew_hbm, tab_hbm, sh_agg, rowb, colb,
                     ewb, xrows, gsem, wid, scale_with_dis, dis_vmem):
    my_chunks = (NCHUNKS - wid + 31) // 32

    def chunk_body(i, _):
        cc = wid + i * 32
        off = cc * CH
        pltpu.sync_copy(row_hbm.at[pl.ds(off, CH)], rowb)
        pltpu.sync_copy(col_hbm.at[pl.ds(off, CH)], colb)
        pltpu.sync_copy(ew_hbm.at[pl.ds(off, CH)], ewb)
        pltpu.async_copy(tab_hbm.at[rowb], xrows, gsem).wait()

        def group_body(gidx, _):
            ev = ewb[pl.ds(gidx * 16, 16)]
            if scale_with_dis:
                rv = rowb[pl.ds(gidx * 16, 16)]
                ev = ev * plsc.load_gather(dis_vmem, [rv])
            for j in range(16):
                er = gidx * 16 + j
                s = ev[j]
                for dblk in range(8):
                    xrows[er, pl.ds(dblk * 16, 16)] = (
                        xrows[er, pl.ds(dblk * 16, 16)] * s)
            return 0

        lax.fori_loop(0, CH // 16, group_body, 0)
        pltpu.sync_copy(xrows, sh_agg.at[colb], add=True)
        return 0

    lax.fori_loop(0, my_chunks, chunk_body, 0)


def _writeout_partial(sh_agg, out_hbm, cid, sid):
    base = sid * PADROWS_PER_SUB
    pltpu.sync_copy(sh_agg.at[pl.ds(base, PADROWS_PER_SUB), :],
                    out_hbm.at[cid, pl.ds(base, PADROWS_PER_SUB), :])


def _sc_layer1(row, col, ew, x):
    mesh = plsc.VectorSubcoreMesh(core_axis_name="c", subcore_axis_name="s")

    @functools.partial(
        pl.kernel,
        out_type=[jax.ShapeDtypeStruct((NC, NPAD, D), jnp.float32),
                  jax.ShapeDtypeStruct((NPAD,), jnp.float32)],
        mesh=mesh,
        compiler_params=pltpu.CompilerParams(needs_layout_passes=False),
        scratch_types=[
            pltpu.VMEM_SHARED((NPAD, D), jnp.float32),
            pltpu.VMEM_SHARED((NPAD,), jnp.float32),
            pltpu.VMEM_SHARED((NPAD,), jnp.float32),
            pltpu.VMEM((NPAD,), jnp.float32),
            pltpu.VMEM((PADROWS_PER_SUB,), jnp.float32),
            pltpu.VMEM((PADROWS_PER_SUB,), jnp.float32),
            pltpu.VMEM((CH, D), jnp.float32),
            pltpu.VMEM((CH,), jnp.int32),
            pltpu.VMEM((CH,), jnp.int32),
            pltpu.VMEM((CH,), jnp.float32),
            pltpu.SemaphoreType.DMA,
        ],
    )
    def kern(row_hbm, col_hbm, ew_hbm, x_hbm, p_hbm, dis_hbm,
             sh_agg, sh_deg, sh_dis, dis_vmem, vbuf_a, vbuf_b,
             xrows, rowb, colb, ewb, gsem):
        cid = lax.axis_index("c")
        sid = lax.axis_index("s")
        wid = cid * NS + sid

        _zero_rows(xrows, CH)
        for b in range(PADROWS_PER_SUB // CH):
            pltpu.sync_copy(
                xrows, sh_agg.at[pl.ds(sid * PADROWS_PER_SUB + b * CH, CH), :])
        _zero_vec16(vbuf_a, PADROWS_PER_SUB)
        pltpu.sync_copy(vbuf_a, sh_deg.at[pl.ds(sid * PADROWS_PER_SUB,
                                                PADROWS_PER_SUB)])
        plsc.subcore_barrier()

        nchunks_sc = (NCHUNKS - sid + NS - 1) // NS

        def deg_body(i, _):
            cc = sid + i * NS
            off = cc * CH
            pltpu.sync_copy(col_hbm.at[pl.ds(off, CH)], colb)
            pltpu.sync_copy(ew_hbm.at[pl.ds(off, CH)], ewb)
            pltpu.sync_copy(ewb, sh_deg.at[colb], add=True)
            return 0

        lax.fori_loop(0, nchunks_sc, deg_body, 0)
        plsc.subcore_barrier()

        nbase = sid * PADROWS_PER_SUB
        pltpu.sync_copy(sh_deg.at[pl.ds(nbase, PADROWS_PER_SUB)], vbuf_a)

        def dis_body(i, _):
            dv = vbuf_a[pl.ds(i * 16, 16)] + 1.0
            vbuf_b[pl.ds(i * 16, 16)] = _rsqrt16(dv)
            return 0

        lax.fori_loop(0, PADROWS_PER_SUB // 16, dis_body, 0)
        pltpu.sync_copy(vbuf_b, sh_dis.at[pl.ds(nbase, PADROWS_PER_SUB)])

        @pl.when(cid == 0)
        def _():
            pltpu.sync_copy(vbuf_b, dis_hbm.at[pl.ds(nbase, PADROWS_PER_SUB)])

        plsc.subcore_barrier()

        pltpu.sync_copy(sh_dis, dis_vmem)
        _aggregate_edges(row_hbm, col_hbm, ew_hbm, x_hbm, sh_agg, rowb, colb,
                         ewb, xrows, gsem, wid, True, dis_vmem)
        plsc.subcore_barrier()

        _writeout_partial(sh_agg, p_hbm, cid, sid)

    return kern(row, col, ew, x)


def _sc_layer2(row, col, ew, g):
    mesh = plsc.VectorSubcoreMesh(core_axis_name="c", subcore_axis_name="s")

    @functools.partial(
        pl.kernel,
        out_type=[jax.ShapeDtypeStruct((NC, NPAD, D), jnp.float32)],
        mesh=mesh,
        compiler_params=pltpu.CompilerParams(needs_layout_passes=False),
        scratch_types=[
            pltpu.VMEM_SHARED((NPAD, D), jnp.float32),
            pltpu.VMEM((CH, D), jnp.float32),
            pltpu.VMEM((CH,), jnp.int32),
            pltpu.VMEM((CH,), jnp.int32),
            pltpu.VMEM((CH,), jnp.float32),
            pltpu.SemaphoreType.DMA,
        ],
    )
    def kern(row_hbm, col_hbm, ew_hbm, g_hbm, q_hbm,
             sh_agg, xrows, rowb, colb, ewb, gsem):
        cid = lax.axis_index("c")
        sid = lax.axis_index("s")
        wid = cid * NS + sid

        _zero_rows(xrows, CH)
        for b in range(PADROWS_PER_SUB // CH):
            pltpu.sync_copy(
                xrows, sh_agg.at[pl.ds(sid * PADROWS_PER_SUB + b * CH, CH), :])
        plsc.subcore_barrier()

        _aggregate_edges(row_hbm, col_hbm, ew_hbm, g_hbm, sh_agg, rowb, colb,
                         ewb, xrows, gsem, wid, False, None)
        plsc.subcore_barrier()

        _writeout_partial(sh_agg, q_hbm, cid, sid)

    return kern(row, col, ew, g)[0]


BLK = 1000


def _tc_mid_body(x, p0, p1, dis, W1, Wr1, W2, Wr2, b1, br1, b2, br2,
                 g_o, base_o):
    xv = x[...]
    disv = dis[...]
    a = disv * (p0[...] + p1[...]) + (disv * disv) * xv
    h = jnp.maximum(
        jnp.dot(a, W1[...], preferred_element_type=jnp.float32)
        + jnp.dot(xv, Wr1[...], preferred_element_type=jnp.float32)
        + b1[...] + br1[...], 0.0)
    g = disv * jnp.dot(h, W2[...], preferred_element_type=jnp.float32)
    base_o[...] = (jnp.dot(h, Wr2[...], preferred_element_type=jnp.float32)
                   + br2[...] + b2[...] + disv * g)
    g_o[...] = g


def _tc_mid(x, p0, p1, dis, W1, Wr1, W2, Wr2, b1, br1, b2, br2):
    nblk = N // BLK
    rows = lambda i: (i, 0)
    whole = lambda i: (0, 0)
    return pl.pallas_call(
        _tc_mid_body,
        grid=(nblk,),
        in_specs=[
            pl.BlockSpec((BLK, D), rows),
            pl.BlockSpec((BLK, D), rows),
            pl.BlockSpec((BLK, D), rows),
            pl.BlockSpec((BLK, 1), rows),
            pl.BlockSpec((D, 1024), whole),
            pl.BlockSpec((D, 1024), whole),
            pl.BlockSpec((1024, D), whole),
            pl.BlockSpec((1024, D), whole),
            pl.BlockSpec((1, 1024), whole),
            pl.BlockSpec((1, 1024), whole),
            pl.BlockSpec((1, D), whole),
            pl.BlockSpec((1, D), whole),
        ],
        out_specs=[pl.BlockSpec((BLK, D), rows),
                   pl.BlockSpec((BLK, D), rows)],
        out_shape=[jax.ShapeDtypeStruct((N, D), jnp.float32),
                   jax.ShapeDtypeStruct((N, D), jnp.float32)],
    )(x, p0, p1, dis, W1, Wr1, W2, Wr2, b1, br1, b2, br2)


def _tc_final_body(base, q0, q1, dis, out_o):
    out_o[...] = base[...] + dis[...] * (q0[...] + q1[...])


def _tc_final(base, q0, q1, dis):
    nblk = N // BLK
    rows = lambda i: (i, 0)
    return pl.pallas_call(
        _tc_final_body,
        grid=(nblk,),
        in_specs=[pl.BlockSpec((BLK, D), rows),
                  pl.BlockSpec((BLK, D), rows),
                  pl.BlockSpec((BLK, D), rows),
                  pl.BlockSpec((BLK, 1), rows)],
        out_specs=pl.BlockSpec((BLK, D), rows),
        out_shape=jax.ShapeDtypeStruct((N, D), jnp.float32),
    )(base, q0, q1, dis)


def kernel(x, edge_index, edge_attr, W1, b1, W2, b2, Wr1, br1, Wr2, br2):
    row = edge_index[0]
    col = edge_index[1]
    ew = edge_attr

    p, dis_pad = _sc_layer1(row, col, ew, x)
    dis = dis_pad[:N].reshape(N, 1)

    g, base = _tc_mid(x, p[0, :N], p[1, :N], dis,
                      W1, Wr1, W2, Wr2,
                      b1.reshape(1, -1), br1.reshape(1, -1),
                      b2.reshape(1, -1), br2.reshape(1, -1))

    q = _sc_layer2(row, col, ew, g)

    return _tc_final(base, q[0, :N], q[1, :N], dis)
